# baseline (speedup 1.0000x reference)
import functools

import jax
import jax.numpy as jnp
from jax import lax
from jax.experimental import pallas as pl
from jax.experimental.pallas import tpu as pltpu
from jax.experimental.pallas import tpu_sc as plsc

BATCH = 16384
HIST = 50
D = 32
B = BATCH * HIST
NC, NS = 2, 16
NW = NC * NS
COLS_PER_W = BATCH // NW
LOOK_PER_W = HIST * COLS_PER_W
CHUNK = COLS_PER_W
GR = CHUNK // D
NCH = HIST

_mesh = plsc.VectorSubcoreMesh(core_axis_name="c", subcore_axis_name="s")


@functools.partial(
    pl.kernel,
    out_type=jax.ShapeDtypeStruct((B,), jnp.int32),
    mesh=_mesh,
    scratch_types=[
        pltpu.VMEM((8, 128), jnp.int32),
        pltpu.SemaphoreType.DMA,
    ],
)
def _detile_kernel(xt_hbm, xf_hbm, v, sem):
    wid = lax.axis_index("s") * NC + lax.axis_index("c")
    col0 = wid * COLS_PER_W
    for c in range(4):
        col = col0 + c * 128
        for ti in range(7):
            nr = 8 if ti < 6 else 2
            pltpu.sync_copy(
                xt_hbm.at[pl.ds(ti * 8, nr), pl.ds(col, 128)],
                v.at[pl.ds(0, nr)],
            )
            copies = [
                pltpu.async_copy(
                    v.at[r],
                    xf_hbm.at[pl.ds((ti * 8 + r) * BATCH + col, 128)],
                    sem,
                )
                for r in range(nr)
            ]
            for cp in copies:
                cp.wait()


@functools.partial(
    pl.kernel,
    out_type=jax.ShapeDtypeStruct((B, D), jnp.float32),
    mesh=_mesh,
    scratch_types=[
        pltpu.VMEM((LOOK_PER_W,), jnp.int32),
        pltpu.VMEM((CHUNK, D), jnp.float32),
        pltpu.VMEM((CHUNK, D), jnp.float32),
        pltpu.SemaphoreType.DMA,
        pltpu.SemaphoreType.DMA,
        pltpu.SemaphoreType.DMA,
        pltpu.SemaphoreType.DMA,
    ],
    compiler_params=pltpu.CompilerParams(use_tc_tiling_on_sc=False),
)
def _gather_kernel(xf_hbm, table_hbm, out_hbm, idx_v, rows0, rows1,
                   sg0, sg1, ss0, ss1):
    wid = lax.axis_index("s") * NC + lax.axis_index("c")
    col0 = wid * COLS_PER_W
    rows_v = (rows0, rows1)
    sg = (sg0, sg1)
    ss = (ss0, ss1)

    def fire(t, b):
        for j in range(GR):
            pltpu.async_copy(
                table_hbm.at[idx_v.at[pl.ds(t * CHUNK + j * D, D)]],
                rows_v[b].at[pl.ds(j * D, D)],
                sg[b],
            )

    def drain_gather(b):
        pltpu.make_async_copy(
            table_hbm.at[pl.ds(0, CHUNK)], rows_v[b], sg[b]).wait()

    def store(t, b):
        pltpu.async_copy(
            rows_v[b],
            out_hbm.at[pl.ds(t * BATCH + col0, CHUNK)],
            ss[b],
        )

    def drain_store(b):
        pltpu.make_async_copy(
            rows_v[b], out_hbm.at[pl.ds(0, CHUNK)], ss[b]).wait()

    stage = [
        pltpu.async_copy(
            xf_hbm.at[pl.ds(h * BATCH + col0, COLS_PER_W)],
            idx_v.at[pl.ds(h * COLS_PER_W, COLS_PER_W)],
            sg0,
        )
        for h in range(HIST)
    ]
    for cp in stage:
        cp.wait()
    fire(0, 0)

    def body(i, carry):
        t = 2 * i
        drain_gather(0)

        @pl.when(i >= 1)
        def _():
            drain_store(1)

        fire(t + 1, 1)
        store(t, 0)
        drain_gather(1)
        drain_store(0)

        @pl.when(t + 2 < NCH)
        def _():
            fire(t + 2, 0)

        store(t + 1, 1)
        return carry

    lax.fori_loop(0, NCH // 2, body, 0)

    drain_store(1)


def kernel(x, W):
    xf = _detile_kernel(jnp.transpose(x).astype(jnp.int32))
    out = _gather_kernel(xf, W)
    return jnp.transpose(out.reshape(HIST, BATCH, D), (1, 0, 2))


# speedup vs baseline: 1.0019x; 1.0019x over previous
import functools

import jax
import jax.numpy as jnp
from jax import lax
from jax.experimental import pallas as pl
from jax.experimental.pallas import tpu as pltpu
from jax.experimental.pallas import tpu_sc as plsc

BATCH = 16384
HIST = 50
D = 32
B = BATCH * HIST
NC, NS = 2, 16
NW = NC * NS
COLS_PER_W = BATCH // NW
LOOK_PER_W = HIST * COLS_PER_W
CHUNK = COLS_PER_W
GR = CHUNK // D
NCH = HIST

_mesh = plsc.VectorSubcoreMesh(core_axis_name="c", subcore_axis_name="s")


@functools.partial(
    pl.kernel,
    out_type=jax.ShapeDtypeStruct((B,), jnp.int32),
    mesh=_mesh,
    scratch_types=[
        pltpu.VMEM((8, 128), jnp.int32),
        pltpu.SemaphoreType.DMA,
    ],
)
def _detile_kernel(xt_hbm, xf_hbm, v, sem):
    wid = lax.axis_index("s") * NC + lax.axis_index("c")
    col0 = wid * COLS_PER_W
    for c in range(4):
        col = col0 + c * 128
        for ti in range(7):
            nr = 8 if ti < 6 else 2
            pltpu.sync_copy(
                xt_hbm.at[pl.ds(ti * 8, nr), pl.ds(col, 128)],
                v.at[pl.ds(0, nr)],
            )
            copies = [
                pltpu.async_copy(
                    v.at[r],
                    xf_hbm.at[pl.ds((ti * 8 + r) * BATCH + col, 128)],
                    sem,
                )
                for r in range(nr)
            ]
            for cp in copies:
                cp.wait()


@functools.partial(
    pl.kernel,
    out_type=jax.ShapeDtypeStruct((HIST, BATCH, D), jnp.float32),
    mesh=_mesh,
    scratch_types=[
        pltpu.VMEM((LOOK_PER_W,), jnp.int32),
        pltpu.VMEM((CHUNK, D), jnp.float32),
        pltpu.VMEM((CHUNK, D), jnp.float32),
        pltpu.SemaphoreType.DMA,
        pltpu.SemaphoreType.DMA,
        pltpu.SemaphoreType.DMA,
        pltpu.SemaphoreType.DMA,
    ],
    compiler_params=pltpu.CompilerParams(use_tc_tiling_on_sc=False),
)
def _gather_kernel(xf_hbm, table_hbm, out_hbm, idx_v, rows0, rows1,
                   sg0, sg1, ss0, ss1):
    wid = lax.axis_index("s") * NC + lax.axis_index("c")
    col0 = wid * COLS_PER_W
    rows_v = (rows0, rows1)
    sg = (sg0, sg1)
    ss = (ss0, ss1)

    def fire(t, b):
        for j in range(GR):
            pltpu.async_copy(
                table_hbm.at[idx_v.at[pl.ds(t * CHUNK + j * D, D)]],
                rows_v[b].at[pl.ds(j * D, D)],
                sg[b],
            )

    def drain_gather(b):
        pltpu.make_async_copy(
            table_hbm.at[pl.ds(0, CHUNK)], rows_v[b], sg[b]).wait()

    def store(t, b):
        pltpu.async_copy(
            rows_v[b],
            out_hbm.at[t, pl.ds(col0, CHUNK)],
            ss[b],
        )

    def drain_store(b):
        pltpu.make_async_copy(
            rows_v[b], out_hbm.at[0, pl.ds(0, CHUNK)], ss[b]).wait()

    stage = [
        pltpu.async_copy(
            xf_hbm.at[pl.ds(h * BATCH + col0, COLS_PER_W)],
            idx_v.at[pl.ds(h * COLS_PER_W, COLS_PER_W)],
            sg0,
        )
        for h in range(HIST)
    ]
    for cp in stage:
        cp.wait()
    fire(0, 0)

    def body(i, carry):
        t = 2 * i
        drain_gather(0)

        @pl.when(i >= 1)
        def _():
            drain_store(1)

        fire(t + 1, 1)
        store(t, 0)
        drain_gather(1)
        drain_store(0)

        @pl.when(t + 2 < NCH)
        def _():
            fire(t + 2, 0)

        store(t + 1, 1)
        return carry

    lax.fori_loop(0, NCH // 2, body, 0)

    drain_store(1)


def kernel(x, W):
    xf = _detile_kernel(jnp.transpose(x).astype(jnp.int32))
    out = _gather_kernel(xf, W)
    return jnp.transpose(out, (1, 0, 2))
